# Initial kernel scaffold; baseline (speedup 1.0000x reference)
import jax
import jax.numpy as jnp
from jax.experimental import pallas as pl


def kernel(edge_attr, edge_src, edge_dst, species):
    raise NotImplementedError("write your pallas kernel here")


# speedup vs baseline: 3.9789x; 3.9789x over previous
import functools

import jax
import jax.numpy as jnp
from jax import lax
from jax.experimental import pallas as pl
from jax.experimental.pallas import tpu as pltpu
from jax.experimental.pallas import tpu_sc as plsc

CHUNK = 128
LANES = 16


def _body(nat, n_chunks, d_core, n_cores, n_sub,
          edge_hbm, src_hbm, dst_hbm, out_hbm,
          acc_src, acc_dst, rows_v, idx_v, zbuf, fa, fb):
    c = lax.axis_index("c")
    s = lax.axis_index("s")
    rows_per_sub = nat // n_sub
    col0 = c * d_core

    zrows = zbuf.shape[0]
    n_z = zrows * d_core // LANES

    def zero_store(t, _):
        i = t // (d_core // LANES)
        k = t % (d_core // LANES)
        zbuf[i, pl.ds(k * LANES, LANES)] = jnp.zeros((LANES,), jnp.float32)
        return 0

    lax.fori_loop(0, n_z, zero_store, 0)
    for b in range(rows_per_sub // zrows):
        base = s * rows_per_sub + b * zrows
        pltpu.sync_copy(zbuf, acc_src.at[pl.ds(base, zrows)])
        pltpu.sync_copy(zbuf, acc_dst.at[pl.ds(base, zrows)])
    plsc.subcore_barrier()

    n_base = n_chunks // n_sub
    n_rem = n_chunks % n_sub
    cnt = n_base + jnp.where(s < n_rem, 1, 0)
    start = s * n_base + jnp.minimum(s, n_rem)

    def chunk_body(j, _):
        ch = start + j
        pltpu.sync_copy(src_hbm.at[ch], idx_v.at[0])
        pltpu.sync_copy(dst_hbm.at[ch], idx_v.at[1])
        pltpu.sync_copy(
            edge_hbm.at[pl.ds(ch * CHUNK, CHUNK), pl.ds(col0, d_core)], rows_v)
        pltpu.sync_copy(rows_v, acc_src.at[idx_v.at[0]], add=True)
        pltpu.sync_copy(rows_v, acc_dst.at[idx_v.at[1]], add=True)
        return 0

    lax.fori_loop(0, cnt, chunk_body, 0)
    plsc.subcore_barrier()

    frows = fa.shape[0]
    n_f = frows * d_core // LANES

    def sub_body(t, _):
        i = t // (d_core // LANES)
        k = t % (d_core // LANES)
        sl = pl.ds(k * LANES, LANES)
        fa[i, sl] = fa[i, sl] - fb[i, sl]
        return 0

    for b in range(rows_per_sub // frows):
        row0 = s * rows_per_sub + b * frows
        pltpu.sync_copy(acc_src.at[pl.ds(row0, frows)], fa)
        pltpu.sync_copy(acc_dst.at[pl.ds(row0, frows)], fb)
        lax.fori_loop(0, n_f, sub_body, 0)
        pltpu.sync_copy(fa, out_hbm.at[pl.ds(row0, frows),
                                       pl.ds(col0, d_core)])


def kernel(edge_attr, edge_src, edge_dst, species):
    nat = species.shape[0]
    n_edges, d_feat = edge_attr.shape
    info = plsc.get_sparse_core_info()
    n_cores, n_sub = info.num_cores, info.num_subcores
    d_core = d_feat // n_cores
    n_chunks = n_edges // CHUNK
    rows_per_sub = nat // n_sub
    zrows = 125

    src2d = edge_src.reshape(n_chunks, CHUNK)
    dst2d = edge_dst.reshape(n_chunks, CHUNK)

    mesh = plsc.VectorSubcoreMesh(core_axis_name="c", subcore_axis_name="s")
    body = functools.partial(_body, nat, n_chunks, d_core, n_cores, n_sub)
    k = pl.kernel(
        body,
        out_type=jax.ShapeDtypeStruct((nat, d_feat), jnp.float32),
        mesh=mesh,
        scratch_types=[
            pltpu.VMEM_SHARED((nat, d_core), jnp.float32),
            pltpu.VMEM_SHARED((nat, d_core), jnp.float32),
            pltpu.VMEM((CHUNK, d_core), jnp.float32),
            pltpu.VMEM((2, CHUNK), jnp.int32),
            pltpu.VMEM((zrows, d_core), jnp.float32),
            pltpu.VMEM((zrows, d_core), jnp.float32),
            pltpu.VMEM((zrows, d_core), jnp.float32),
        ],
        compiler_params=pltpu.CompilerParams(use_tc_tiling_on_sc=False),
    )
    return k(edge_attr, src2d, dst2d)


# speedup vs baseline: 10.1199x; 2.5434x over previous
import functools

import jax
import jax.numpy as jnp
from jax import lax
from jax.experimental import pallas as pl
from jax.experimental.pallas import tpu as pltpu
from jax.experimental.pallas import tpu_sc as plsc

CHUNK = 128
NSLOT = 3
LANES = 16


def _body(nat, n_chunks, d_core, n_cores, n_sub,
          edge_hbm, src_hbm, dst_hbm, out_hbm,
          acc_src, acc_dst, rows0, rows1, rows2, idx0, idx1, idx2,
          fa, fb, sem_l0, sem_l1, sem_l2, sem_s):
    c = lax.axis_index("c")
    s = lax.axis_index("s")
    rows_per_sub = nat // n_sub
    col0 = c * d_core

    rows_b = (rows0, rows1, rows2)
    idx_b = (idx0, idx1, idx2)
    sem_l = (sem_l0, sem_l1, sem_l2)

    zrows = rows0.shape[0]
    n_z = zrows * d_core // LANES

    def zero_store(t, _):
        i = t // (d_core // LANES)
        k = t % (d_core // LANES)
        rows0[i, pl.ds(k * LANES, LANES)] = jnp.zeros((LANES,), jnp.float32)
        return 0

    lax.fori_loop(0, n_z, zero_store, 0)
    zsrc = rows0.at[pl.ds(0, 125)]
    for b in range(rows_per_sub // 125):
        base = s * rows_per_sub + b * 125
        pltpu.sync_copy(zsrc, acc_src.at[pl.ds(base, 125)])
        pltpu.sync_copy(zsrc, acc_dst.at[pl.ds(base, 125)])
    plsc.subcore_barrier()

    n_base = n_chunks // n_sub
    n_rem = n_chunks % n_sub
    cnt = n_base + jnp.where(s < n_rem, 1, 0)
    start = s * n_base + jnp.minimum(s, n_rem)
    t_static = n_base + (1 if n_rem else 0)

    def load_args(gi, b):
        ch = start + gi
        return (
            (src_hbm.at[ch], idx_b[b].at[0]),
            (dst_hbm.at[ch], idx_b[b].at[1]),
            (edge_hbm.at[pl.ds(ch * CHUNK, CHUNK),
                         pl.ds(col0, d_core)], rows_b[b]),
        )

    def start_loads(gi, b):
        for src, dst in load_args(gi, b):
            pltpu.async_copy(src, dst, sem_l[b])

    def wait_loads(gi, b):
        for src, dst in load_args(gi, b):
            pltpu.make_async_copy(src, dst, sem_l[b]).wait()

    def drain_scatters(gi, b):
        pltpu.make_async_copy(rows_b[b], acc_src.at[idx_b[b].at[0]], sem_s).wait()
        pltpu.make_async_copy(rows_b[b], acc_dst.at[idx_b[b].at[1]], sem_s).wait()

    start_loads(0, 0)

    @pl.when(1 < cnt)
    def _():
        start_loads(1, 1)

    def loop_body(go, _):
        for b in range(NSLOT):
            gi = go * NSLOT + b
            pb = (b + NSLOT - 1) % NSLOT

            @pl.when((gi >= 1) & (gi <= cnt))
            def _():
                drain_scatters(gi - 1, pb)

            @pl.when(gi + 2 < cnt)
            def _():
                start_loads(gi + 2, pb)

            @pl.when(gi < cnt)
            def _():
                wait_loads(gi, b)
                pltpu.async_copy(
                    rows_b[b], acc_src.at[idx_b[b].at[0]], sem_s, add=True)
                pltpu.async_copy(
                    rows_b[b], acc_dst.at[idx_b[b].at[1]], sem_s, add=True)

        return 0

    lax.fori_loop(0, (t_static + NSLOT) // NSLOT, loop_body, 0)
    plsc.subcore_barrier()

    frows = fa.shape[0]
    n_f = frows * d_core // LANES

    def sub_body(t, _):
        i = t // (d_core // LANES)
        k = t % (d_core // LANES)
        sl = pl.ds(k * LANES, LANES)
        fa[i, sl] = fa[i, sl] - fb[i, sl]
        return 0

    for b in range(rows_per_sub // frows):
        row0 = s * rows_per_sub + b * frows
        pltpu.sync_copy(acc_src.at[pl.ds(row0, frows)], fa)
        pltpu.sync_copy(acc_dst.at[pl.ds(row0, frows)], fb)
        lax.fori_loop(0, n_f, sub_body, 0)
        pltpu.sync_copy(fa, out_hbm.at[pl.ds(row0, frows),
                                       pl.ds(col0, d_core)])


def kernel(edge_attr, edge_src, edge_dst, species):
    nat = species.shape[0]
    n_edges, d_feat = edge_attr.shape
    info = plsc.get_sparse_core_info()
    n_cores, n_sub = info.num_cores, info.num_subcores
    d_core = d_feat // n_cores
    n_chunks = n_edges // CHUNK
    frows = 125

    src2d = edge_src.reshape(n_chunks, CHUNK)
    dst2d = edge_dst.reshape(n_chunks, CHUNK)

    mesh = plsc.VectorSubcoreMesh(core_axis_name="c", subcore_axis_name="s")
    body = functools.partial(_body, nat, n_chunks, d_core, n_cores, n_sub)
    k = pl.kernel(
        body,
        out_type=jax.ShapeDtypeStruct((nat, d_feat), jnp.float32),
        mesh=mesh,
        scratch_types=[
            pltpu.VMEM_SHARED((nat, d_core), jnp.float32),
            pltpu.VMEM_SHARED((nat, d_core), jnp.float32),
            pltpu.VMEM((CHUNK, d_core), jnp.float32),
            pltpu.VMEM((CHUNK, d_core), jnp.float32),
            pltpu.VMEM((CHUNK, d_core), jnp.float32),
            pltpu.VMEM((2, CHUNK), jnp.int32),
            pltpu.VMEM((2, CHUNK), jnp.int32),
            pltpu.VMEM((2, CHUNK), jnp.int32),
            pltpu.VMEM((frows, d_core), jnp.float32),
            pltpu.VMEM((frows, d_core), jnp.float32),
            pltpu.SemaphoreType.DMA,
            pltpu.SemaphoreType.DMA,
            pltpu.SemaphoreType.DMA,
            pltpu.SemaphoreType.DMA,
        ],
        compiler_params=pltpu.CompilerParams(use_tc_tiling_on_sc=False),
    )
    return k(edge_attr, src2d, dst2d)
